# baseline (speedup 1.0000x reference)
import functools

import jax
import jax.numpy as jnp
from jax import lax
from jax.experimental import pallas as pl
from jax.experimental.pallas import tpu as pltpu
from jax.experimental.pallas import tpu_sc as plsc

DIGITS = 8
HIDDEN = 128
BATCH = 16384
NLANES = 16
NCORES = 2
NSUB = 16
NW = NCORES * NSUB
BPW = BATCH // NW
HREGS = HIDDEN // NLANES
NPAIR = DIGITS // 2


def _sc_body(nums_hbm, emb_hbm, out_hbm, emb_v, pair_v, nums_v, out_v):
    wid = lax.axis_index("s") * NCORES + lax.axis_index("c")
    base = wid * BPW
    pltpu.sync_copy(emb_hbm, emb_v)
    pltpu.sync_copy(nums_hbm.at[pl.ds(base, BPW)], nums_v)

    def build(v, c):
        d1 = lax.rem(v, 10)
        d2 = lax.div(v, 10)
        for p in range(NPAIR):
            a = 20 * p + d1
            b = 20 * p + 10 + d2
            r = 100 * p + v
            for h in range(HREGS):
                s = pl.ds(h * NLANES, NLANES)
                pair_v[pl.ds(r * HIDDEN + h * NLANES, NLANES)] = (
                    emb_v[a, s] + emb_v[b, s])
        return c

    lax.fori_loop(0, 100, build, 0)

    iota = lax.iota(jnp.int32, NLANES)

    def body(g, c):
        nv = nums_v[pl.ds(g * NLANES, NLANES)]
        idxs = []
        n = nv
        for p in range(NPAIR):
            idxs.append((lax.rem(n, 100) + 100 * p) * HIDDEN)
            n = lax.div(n, 100)
        ov = (iota + g * NLANES) * HIDDEN
        for h in range(HIDDEN):
            acc = plsc.load_gather(pair_v, [idxs[0]])
            for p in range(1, NPAIR):
                acc = acc + plsc.load_gather(pair_v, [idxs[p]])
            plsc.store_scatter(out_v, [ov], acc)
            if h != HIDDEN - 1:
                for p in range(NPAIR):
                    idxs[p] = idxs[p] + 1
                ov = ov + 1
        return c

    lax.fori_loop(0, BPW // NLANES, body, 0)
    pltpu.sync_copy(out_v, out_hbm.at[pl.ds(base * HIDDEN, BPW * HIDDEN)])


@functools.partial(jax.jit, static_argnames=())
def kernel(nums, emb):
    nums = nums.astype(jnp.int32)
    mesh = plsc.VectorSubcoreMesh(core_axis_name="c", subcore_axis_name="s")
    f = functools.partial(
        pl.kernel,
        out_type=jax.ShapeDtypeStruct((BATCH * HIDDEN,), jnp.float32),
        mesh=mesh,
        compiler_params=pltpu.CompilerParams(needs_layout_passes=False),
        scratch_types=[
            pltpu.VMEM((DIGITS * 10, HIDDEN), jnp.float32),
            pltpu.VMEM((NPAIR * 100 * HIDDEN,), jnp.float32),
            pltpu.VMEM((BPW,), jnp.int32),
            pltpu.VMEM((BPW * HIDDEN,), jnp.float32),
        ],
    )(_sc_body)
    return f(nums, emb).reshape(BATCH, HIDDEN)


# speedup vs baseline: 2.8945x; 2.8945x over previous
import functools

import jax
import jax.numpy as jnp
from jax import lax
from jax.experimental import pallas as pl
from jax.experimental.pallas import tpu as pltpu
from jax.experimental.pallas import tpu_sc as plsc

DIGITS = 8
HIDDEN = 128
BATCH = 16384
NLANES = 16
NCORES = 2
NSUB = 16
NW = NCORES * NSUB
BPW = BATCH // NW
HREGS = HIDDEN // NLANES
NPAIR = DIGITS // 2


def _sc_body(nums_hbm, emb_hbm, out_hbm, emb_v, pair_v, nums_v, out_v):
    wid = lax.axis_index("s") * NCORES + lax.axis_index("c")
    base = wid * BPW
    pltpu.sync_copy(emb_hbm, emb_v)
    pltpu.sync_copy(nums_hbm.at[pl.ds(base, BPW)], nums_v)

    def build(v, c):
        d1 = lax.rem(v, 10)
        d2 = lax.div(v, 10)
        for p in range(NPAIR):
            a = 20 * p + d1
            b = 20 * p + 10 + d2
            r = 100 * p + v
            for h in range(HREGS):
                s = pl.ds(h * NLANES, NLANES)
                pair_v[r, s] = emb_v[a, s] + emb_v[b, s]
        return c

    lax.fori_loop(0, 100, build, 0)

    def body(g, c):
        nv = nums_v[pl.ds(g * NLANES, NLANES)]
        rvecs = []
        n = nv
        for p in range(NPAIR):
            rvecs.append(lax.rem(n, 100) + 100 * p)
            n = lax.div(n, 100)
        for k in range(NLANES):
            rows = [rvecs[p][k] for p in range(NPAIR)]
            j = g * NLANES + k
            for h in range(HREGS):
                s = pl.ds(h * NLANES, NLANES)
                acc = pair_v[rows[0], s]
                for p in range(1, NPAIR):
                    acc = acc + pair_v[rows[p], s]
                out_v[j, s] = acc
        return c

    lax.fori_loop(0, BPW // NLANES, body, 0)
    pltpu.sync_copy(out_v, out_hbm.at[pl.ds(base, BPW)])


@functools.partial(jax.jit, static_argnames=())
def kernel(nums, emb):
    nums = nums.astype(jnp.int32)
    mesh = plsc.VectorSubcoreMesh(core_axis_name="c", subcore_axis_name="s")
    f = functools.partial(
        pl.kernel,
        out_type=jax.ShapeDtypeStruct((BATCH, HIDDEN), jnp.float32),
        mesh=mesh,
        scratch_types=[
            pltpu.VMEM((DIGITS * 10, HIDDEN), jnp.float32),
            pltpu.VMEM((NPAIR * 100, HIDDEN), jnp.float32),
            pltpu.VMEM((BPW,), jnp.int32),
            pltpu.VMEM((BPW, HIDDEN), jnp.float32),
        ],
    )(_sc_body)
    return f(nums, emb)


# speedup vs baseline: 3.8147x; 1.3179x over previous
import functools

import jax
import jax.numpy as jnp
from jax import lax
from jax.experimental import pallas as pl
from jax.experimental.pallas import tpu as pltpu
from jax.experimental.pallas import tpu_sc as plsc

DIGITS = 8
HIDDEN = 128
BATCH = 16384
NLANES = 16
NCORES = 2
NSUB = 16
NW = NCORES * NSUB
BPW = BATCH // NW
NPAIR = DIGITS // 2
NCHUNK = HIDDEN // (2 * NLANES)


def _sc_body(nums_hbm, emb_hbm, out_hbm, emb_v, pairb_v, nums_v, out_v):
    wid = lax.axis_index("s") * NCORES + lax.axis_index("c")
    base = wid * BPW
    pltpu.sync_copy(emb_hbm, emb_v)
    pltpu.sync_copy(nums_hbm.at[pl.ds(base, BPW)], nums_v)

    def build(v, c):
        d1 = lax.rem(v, 10)
        d2 = lax.div(v, 10)
        for p in range(NPAIR):
            a = 20 * p + d1
            b = 20 * p + 10 + d2
            r = 100 * p + v
            for ch in range(NCHUNK):
                s0 = pl.ds(ch * 32, NLANES)
                s1 = pl.ds(ch * 32 + NLANES, NLANES)
                acc0 = emb_v[a, s0] + emb_v[b, s0]
                acc1 = emb_v[a, s1] + emb_v[b, s1]
                pairb_v[r, pl.ds(ch * 32, 32)] = plsc.pack(
                    acc0, acc1, format=plsc.PackFormat.INTERLEAVED)
        return c

    lax.fori_loop(0, 100, build, 0)

    def body(g, c):
        nv = nums_v[pl.ds(g * NLANES, NLANES)]
        rvecs = []
        n = nv
        for p in range(NPAIR):
            rvecs.append(lax.rem(n, 100) + 100 * p)
            n = lax.div(n, 100)
        for k in range(NLANES):
            rows = [rvecs[p][k] for p in range(NPAIR)]
            j = g * NLANES + k
            for ch in range(NCHUNK):
                acc0, acc1 = plsc.unpack(
                    pairb_v[rows[0], pl.ds(ch * 32, 32)],
                    format=plsc.PackFormat.INTERLEAVED)
                for p in range(1, NPAIR):
                    a, b = plsc.unpack(
                        pairb_v[rows[p], pl.ds(ch * 32, 32)],
                        format=plsc.PackFormat.INTERLEAVED)
                    acc0 = acc0 + a
                    acc1 = acc1 + b
                out_v[j, pl.ds(ch * 32, NLANES)] = acc0
                out_v[j, pl.ds(ch * 32 + NLANES, NLANES)] = acc1
        return c

    lax.fori_loop(0, BPW // NLANES, body, 0)
    pltpu.sync_copy(out_v, out_hbm.at[pl.ds(base, BPW)])


@functools.partial(jax.jit, static_argnames=())
def kernel(nums, emb):
    nums = nums.astype(jnp.int32)
    mesh = plsc.VectorSubcoreMesh(core_axis_name="c", subcore_axis_name="s")
    f = functools.partial(
        pl.kernel,
        out_type=jax.ShapeDtypeStruct((BATCH, HIDDEN), jnp.float32),
        mesh=mesh,
        compiler_params=pltpu.CompilerParams(needs_layout_passes=False),
        scratch_types=[
            pltpu.VMEM((DIGITS * 10, HIDDEN), jnp.float32),
            pltpu.VMEM((NPAIR * 100, HIDDEN), jnp.bfloat16),
            pltpu.VMEM((BPW,), jnp.int32),
            pltpu.VMEM((BPW, HIDDEN), jnp.float32),
        ],
    )(_sc_body)
    return f(nums, emb)
